# baseline (speedup 1.0000x reference)
import functools

import jax
import jax.numpy as jnp
from jax.experimental import pallas as pl
from jax.experimental.pallas import tpu as pltpu


def _fused(x_ref, w_ref, emb_ref, b_ref, onehot_ref, out_ref,
           s_acc, logits_acc, bias_acc, *, s_len, n_chunks, n_s):
    k = pl.program_id(0)
    j = pl.program_id(1)

    @pl.when((k == 0) & (j == 0))
    def _init():
        logits_acc[...] = jnp.zeros_like(logits_acc)
        bias_acc[...] = jax.lax.dot_general(
            b_ref[...], emb_ref[...],
            dimension_numbers=(((1,), (1,)), ((), ())),
            preferred_element_type=jnp.float32)

    @pl.when(j == 0)
    def _first_slice():
        s_acc[...] = jnp.sum(x_ref[...], axis=1)

    @pl.when(j != 0)
    def _next_slice():
        s_acc[...] += jnp.sum(x_ref[...], axis=1)

    @pl.when(j == n_s - 1)
    def _chunk_done():
        state_chunk = s_acc[...] * (1.0 / s_len)
        g_chunk = jax.lax.dot_general(
            emb_ref[...], w_ref[...],
            dimension_numbers=(((1,), (0,)), ((), ())),
            preferred_element_type=jnp.float32)
        logits_acc[...] += jax.lax.dot_general(
            state_chunk, g_chunk,
            dimension_numbers=(((1,), (1,)), ((), ())),
            preferred_element_type=jnp.float32)

    @pl.when((k == n_chunks - 1) & (j == n_s - 1))
    def _finish():
        logits = logits_acc[...] + bias_acc[...]
        m = jnp.max(logits, axis=1, keepdims=True)
        lse = jnp.log(jnp.sum(jnp.exp(logits - m), axis=1, keepdims=True)) + m
        logp = logits - lse
        picked = jnp.sum(logp * onehot_ref[...], axis=1, keepdims=True)
        out_ref[...] = jnp.sum(picked, axis=0, keepdims=True) * (
            -0.001 / logits.shape[0])


def kernel(hidden_states, representations, W, b, current_indices,
           current_expert_idx, current_depth):
    B, S, H = hidden_states.shape
    E = representations.shape[0]
    C = 256
    NS = 2
    n = H // C
    sb = S // NS

    emb = jnp.take(representations, current_indices, axis=0)
    onehot = (jax.lax.iota(jnp.int32, E)[None, :]
              == jnp.asarray(current_expert_idx, jnp.int32)).astype(jnp.float32)
    b2 = b.reshape(1, H)

    out = pl.pallas_call(
        functools.partial(_fused, s_len=S, n_chunks=n, n_s=NS),
        grid=(n, NS),
        in_specs=[
            pl.BlockSpec((B, sb, C), lambda k, j: (0, j, k)),
            pl.BlockSpec((H, C), lambda k, j: (0, k)),
            pl.BlockSpec((E, H), lambda k, j: (0, 0)),
            pl.BlockSpec((1, H), lambda k, j: (0, 0)),
            pl.BlockSpec((1, E), lambda k, j: (0, 0)),
        ],
        out_specs=pl.BlockSpec((1, 1), lambda k, j: (0, 0)),
        out_shape=jax.ShapeDtypeStruct((1, 1), jnp.float32),
        scratch_shapes=[pltpu.VMEM((B, C), jnp.float32),
                        pltpu.VMEM((B, E), jnp.float32),
                        pltpu.VMEM((1, E), jnp.float32)],
    )(hidden_states, W, emb, b2, onehot)
    return out[0, 0]
